# baseline (speedup 1.0000x reference)
import jax
import jax.numpy as jnp
from jax.experimental import pallas as pl

_B, _D, _K = 4096, 64, 32
_RB = 256
_G = _B // _RB
_MIN_BIN = 0.001
_MIN_DER = 0.001


def _softplus(v):
    return jnp.maximum(v, 0.0) + jnp.log1p(jnp.exp(-jnp.abs(v)))


def _t(a):
    return jax.lax.transpose(a, (1, 0))


def _body(x_ref, cx_ref, cy_ref, uw_ref, uh_ref, ud_ref, out_ref, lad_ref):
    K = _K
    x = _t(x_ref[...])
    cx = _t(cx_ref[...])
    cy = _t(cy_ref[...])
    z = x - cx

    spw = _MIN_BIN + _softplus(_t(uw_ref[...]).reshape(_D, K, _RB))
    sph = _MIN_BIN + _softplus(_t(uh_ref[...]).reshape(_D, K, _RB))
    ud3 = _t(ud_ref[...]).reshape(_D, K + 1, _RB)

    def csum(c):
        for s in (1, 2, 4, 8, 16):
            sh = jnp.concatenate(
                [jnp.zeros((_D, s, _RB), jnp.float32), c[:, :-s, :]], axis=1)
            c = c + sh
        return c

    cw = csum(spw)
    chh = csum(sph)
    totw = cw[:, K - 1:K, :]
    toth = chh[:, K - 1:K, :]
    ecw = jnp.concatenate([jnp.zeros((_D, 1, _RB), jnp.float32), cw],
                          axis=1) - 0.5 * totw
    ech = jnp.concatenate([jnp.zeros((_D, 1, _RB), jnp.float32), chh],
                          axis=1) - 0.5 * toth

    e0 = ecw[:, 0, :]
    eK = ecw[:, K, :]
    ch0 = ech[:, 0, :]
    chK = ech[:, K, :]

    lm = z < e0
    rm = z >= eK
    im = jnp.logical_not(jnp.logical_or(lm, rm))
    zst = jnp.where(im, z, 0.0)
    zst3 = zst[:, None, :]

    ind = (zst3 >= ecw).astype(jnp.float32)
    idx = jnp.sum(ind, axis=1).astype(jnp.int32) - 1
    idx3 = idx[:, None, :]

    k33 = jax.lax.broadcasted_iota(jnp.int32, (_D, K + 1, _RB), 1)
    k32 = jax.lax.broadcasted_iota(jnp.int32, (_D, K, _RB), 1)
    oh_lo33 = k33 == idx3
    oh_hi33 = k33 == idx3 + 1
    oh32 = k32 == idx3

    def gat(mask, arr):
        return jnp.sum(jnp.where(mask, arr, 0.0), axis=1)

    cw_lo = gat(oh_lo33, ecw)
    ch_lo = gat(oh_lo33, ech)
    w_b = gat(oh32, spw)
    h_b = gat(oh32, sph)
    d_lo_raw = gat(oh_lo33, ud3)
    d_hi_raw = gat(oh_hi33, ud3)
    d0_raw = ud3[:, 0, :]
    dK_raw = ud3[:, K, :]

    d_lo = _MIN_DER + _softplus(d_lo_raw)
    d_hi = _MIN_DER + _softplus(d_hi_raw)
    d0 = _MIN_DER + _softplus(d0_raw)
    dK = _MIN_DER + _softplus(dK_raw)

    out_left = (ch0 + cy) - (e0 - z) * d0
    out_right = (z - eK) * dK + (chK + cy)
    lad_left = jnp.log(d0)
    lad_right = jnp.log(dK)

    theta = (zst - cw_lo) / w_b
    tmt = theta * (1.0 - theta)
    delta = h_b / w_b
    numer = h_b * (delta * theta * theta + d_lo * tmt)
    denom = delta + (d_lo + d_hi - 2.0 * delta) * tmt
    out_in = (ch_lo + cy) + numer / denom
    dnum = (delta * delta) * (d_hi * theta * theta + 2.0 * delta * tmt
                              + d_lo * (1.0 - theta) * (1.0 - theta))
    lad_in = jnp.log(dnum) - 2.0 * jnp.log(denom)

    out = jnp.where(lm, out_left, jnp.where(rm, out_right, out_in))
    lad = jnp.where(lm, lad_left, jnp.where(rm, lad_right, lad_in))
    out_ref[...] = _t(out)
    lad_ref[...] = _t(lad)


_NCHUNK = 4
_BC = _B // _NCHUNK


def kernel(inputs, unnormalized_widths, unnormalized_heights,
           unnormalized_derivatives, center_x, center_y):
    espec = pl.BlockSpec((_RB, _D), lambda i: (i, 0))
    kspec = lambda k: pl.BlockSpec((_RB, _D * k), lambda i: (i, 0))

    call = pl.pallas_call(
        _body,
        grid=(_BC // _RB,),
        in_specs=[espec, espec, espec,
                  kspec(_K), kspec(_K), kspec(_K + 1)],
        out_specs=[espec, espec],
        out_shape=[jax.ShapeDtypeStruct((_BC, _D), jnp.float32),
                   jax.ShapeDtypeStruct((_BC, _D), jnp.float32)],
    )

    outs, lads = [], []
    for c in range(_NCHUNK):
        s = slice(c * _BC, (c + 1) * _BC)
        uw = unnormalized_widths[s].reshape(_BC, _D * _K)
        uh = unnormalized_heights[s].reshape(_BC, _D * _K)
        ud = unnormalized_derivatives[s].reshape(_BC, _D * (_K + 1))
        o, l = call(inputs[s], center_x[s], center_y[s], uw, uh, ud)
        outs.append(o)
        lads.append(l)
    return (jnp.concatenate(outs, axis=0), jnp.concatenate(lads, axis=0))


# speedup vs baseline: 1.3281x; 1.3281x over previous
import jax
import jax.numpy as jnp
from jax.experimental import pallas as pl

_B, _D, _K = 4096, 64, 32
_RB = 256
_G = _B // _RB
_MIN_BIN = 0.001
_MIN_DER = 0.001


def _softplus(v):
    return jnp.maximum(v, 0.0) + jnp.log1p(jnp.exp(-jnp.abs(v)))


def _t(a):
    return jax.lax.transpose(a, (1, 0))


def _psum(a):
    p = a.shape[1]
    extra = None
    if p % 2:
        extra = a[:, p - 1, :]
        a = a[:, :p - 1, :]
        p -= 1
    while p > 1:
        h = p // 2
        a = a[:, :h, :] + a[:, h:p, :]
        p = h
    r = a[:, 0, :]
    return r if extra is None else r + extra


def _body(x_ref, cx_ref, cy_ref, uw_ref, uh_ref, ud_ref, out_ref, lad_ref):
    K = _K
    x = _t(x_ref[...])
    cx = _t(cx_ref[...])
    cy = _t(cy_ref[...])
    z = x - cx

    spw = _MIN_BIN + _softplus(_t(uw_ref[...]).reshape(_D, K, _RB))
    sph = _MIN_BIN + _softplus(_t(uh_ref[...]).reshape(_D, K, _RB))

    def csum(c):
        for s in (1, 2, 4, 8, 16):
            sh = jnp.concatenate(
                [jnp.zeros((_D, s, _RB), jnp.float32), c[:, :-s, :]], axis=1)
            c = c + sh
        return c

    cw = csum(spw)
    chh = csum(sph)
    totw = cw[:, K - 1:K, :]
    toth = chh[:, K - 1:K, :]
    ecw = jnp.concatenate([jnp.zeros((_D, 1, _RB), jnp.float32), cw],
                          axis=1) - 0.5 * totw
    ech = jnp.concatenate([jnp.zeros((_D, 1, _RB), jnp.float32), chh],
                          axis=1) - 0.5 * toth

    e0 = ecw[:, 0, :]
    eK = ecw[:, K, :]
    ch0 = ech[:, 0, :]
    chK = ech[:, K, :]

    lm = z < e0
    rm = z >= eK
    im = jnp.logical_not(jnp.logical_or(lm, rm))
    zst = jnp.where(im, z, 0.0)
    zst3 = zst[:, None, :]

    ind = (zst3 >= ecw).astype(jnp.float32)
    idx = _psum(ind).astype(jnp.int32) - 1
    idx3 = idx[:, None, :]

    k33 = jax.lax.broadcasted_iota(jnp.int32, (_D, K + 1, _RB), 1)
    k32 = jax.lax.broadcasted_iota(jnp.int32, (_D, K, _RB), 1)
    oh_lo33 = k33 == idx3
    oh_hi33 = k33 == idx3 + 1
    oh32 = k32 == idx3

    def gat(mask, arr):
        return _psum(jnp.where(mask, arr, 0.0))

    cw_lo = gat(oh_lo33, ecw)
    ch_lo = gat(oh_lo33, ech)
    w_b = gat(oh32, spw)
    h_b = gat(oh32, sph)
    ud3 = _t(ud_ref[...]).reshape(_D, K + 1, _RB)
    d_lo_raw = gat(oh_lo33, ud3)
    d_hi_raw = gat(oh_hi33, ud3)
    d0_raw = ud3[:, 0, :]
    dK_raw = ud3[:, K, :]

    d_lo = _MIN_DER + _softplus(d_lo_raw)
    d_hi = _MIN_DER + _softplus(d_hi_raw)
    d0 = _MIN_DER + _softplus(d0_raw)
    dK = _MIN_DER + _softplus(dK_raw)

    out_left = (ch0 + cy) - (e0 - z) * d0
    out_right = (z - eK) * dK + (chK + cy)
    lad_left = jnp.log(d0)
    lad_right = jnp.log(dK)

    theta = (zst - cw_lo) / w_b
    tmt = theta * (1.0 - theta)
    delta = h_b / w_b
    numer = h_b * (delta * theta * theta + d_lo * tmt)
    denom = delta + (d_lo + d_hi - 2.0 * delta) * tmt
    out_in = (ch_lo + cy) + numer / denom
    dnum = (delta * delta) * (d_hi * theta * theta + 2.0 * delta * tmt
                              + d_lo * (1.0 - theta) * (1.0 - theta))
    lad_in = jnp.log(dnum) - 2.0 * jnp.log(denom)

    out = jnp.where(lm, out_left, jnp.where(rm, out_right, out_in))
    lad = jnp.where(lm, lad_left, jnp.where(rm, lad_right, lad_in))
    out_ref[...] = _t(out)
    lad_ref[...] = _t(lad)


def kernel(inputs, unnormalized_widths, unnormalized_heights,
           unnormalized_derivatives, center_x, center_y):
    uw = unnormalized_widths.reshape(_B, _D * _K)
    uh = unnormalized_heights.reshape(_B, _D * _K)
    ud = unnormalized_derivatives.reshape(_B, _D * (_K + 1))

    espec = pl.BlockSpec((_RB, _D), lambda i: (i, 0))
    kspec = lambda k: pl.BlockSpec((_RB, _D * k), lambda i: (i, 0))

    out, lad = pl.pallas_call(
        _body,
        grid=(_G,),
        in_specs=[espec, espec, espec,
                  kspec(_K), kspec(_K), kspec(_K + 1)],
        out_specs=[espec, espec],
        out_shape=[jax.ShapeDtypeStruct((_B, _D), jnp.float32),
                   jax.ShapeDtypeStruct((_B, _D), jnp.float32)],
    )(inputs, center_x, center_y, uw, uh, ud)
    return out, lad


# speedup vs baseline: 1.3646x; 1.0275x over previous
import jax
import jax.numpy as jnp
from jax.experimental import pallas as pl

_B, _D, _K = 4096, 64, 32
_RB = 512
_G = _B // _RB
_MIN_BIN = 0.001
_MIN_DER = 0.001


def _softplus(v):
    return jnp.maximum(v, 0.0) + jnp.log1p(jnp.exp(-jnp.abs(v)))


def _t(a):
    return jax.lax.transpose(a, (1, 0))


def _psum(a):
    p = a.shape[1]
    extra = None
    if p % 2:
        extra = a[:, p - 1, :]
        a = a[:, :p - 1, :]
        p -= 1
    while p > 1:
        h = p // 2
        a = a[:, :h, :] + a[:, h:p, :]
        p = h
    r = a[:, 0, :]
    return r if extra is None else r + extra


def _body(x_ref, cx_ref, cy_ref, uw_ref, uh_ref, ud_ref, out_ref, lad_ref):
    K = _K
    x = _t(x_ref[...])
    cx = _t(cx_ref[...])
    cy = _t(cy_ref[...])
    z = x - cx

    spw = _MIN_BIN + _softplus(_t(uw_ref[...]).reshape(_D, K, _RB))
    sph = _MIN_BIN + _softplus(_t(uh_ref[...]).reshape(_D, K, _RB))

    def csum(c):
        for s in (1, 2, 4, 8, 16):
            sh = jnp.concatenate(
                [jnp.zeros((_D, s, _RB), jnp.float32), c[:, :-s, :]], axis=1)
            c = c + sh
        return c

    cw = csum(spw)
    chh = csum(sph)
    totw = cw[:, K - 1:K, :]
    toth = chh[:, K - 1:K, :]
    ecw = jnp.concatenate([jnp.zeros((_D, 1, _RB), jnp.float32), cw],
                          axis=1) - 0.5 * totw
    ech = jnp.concatenate([jnp.zeros((_D, 1, _RB), jnp.float32), chh],
                          axis=1) - 0.5 * toth

    e0 = ecw[:, 0, :]
    eK = ecw[:, K, :]
    ch0 = ech[:, 0, :]
    chK = ech[:, K, :]

    lm = z < e0
    rm = z >= eK
    im = jnp.logical_not(jnp.logical_or(lm, rm))
    zst = jnp.where(im, z, 0.0)
    zst3 = zst[:, None, :]

    ind = (zst3 >= ecw).astype(jnp.float32)
    idx = _psum(ind).astype(jnp.int32) - 1
    idx3 = idx[:, None, :]

    k33 = jax.lax.broadcasted_iota(jnp.int32, (_D, K + 1, _RB), 1)
    k32 = jax.lax.broadcasted_iota(jnp.int32, (_D, K, _RB), 1)
    oh_lo33 = k33 == idx3
    oh_hi33 = k33 == idx3 + 1
    oh32 = k32 == idx3

    def gat(mask, arr):
        return _psum(jnp.where(mask, arr, 0.0))

    cw_lo = gat(oh_lo33, ecw)
    ch_lo = gat(oh_lo33, ech)
    w_b = gat(oh32, spw)
    h_b = gat(oh32, sph)
    ud3 = _t(ud_ref[...]).reshape(_D, K + 1, _RB)
    d_lo_raw = gat(oh_lo33, ud3)
    d_hi_raw = gat(oh_hi33, ud3)
    d0_raw = ud3[:, 0, :]
    dK_raw = ud3[:, K, :]

    d_lo = _MIN_DER + _softplus(d_lo_raw)
    d_hi = _MIN_DER + _softplus(d_hi_raw)
    d0 = _MIN_DER + _softplus(d0_raw)
    dK = _MIN_DER + _softplus(dK_raw)

    out_left = (ch0 + cy) - (e0 - z) * d0
    out_right = (z - eK) * dK + (chK + cy)
    lad_left = jnp.log(d0)
    lad_right = jnp.log(dK)

    theta = (zst - cw_lo) / w_b
    tmt = theta * (1.0 - theta)
    delta = h_b / w_b
    numer = h_b * (delta * theta * theta + d_lo * tmt)
    denom = delta + (d_lo + d_hi - 2.0 * delta) * tmt
    out_in = (ch_lo + cy) + numer / denom
    dnum = (delta * delta) * (d_hi * theta * theta + 2.0 * delta * tmt
                              + d_lo * (1.0 - theta) * (1.0 - theta))
    lad_in = jnp.log(dnum) - 2.0 * jnp.log(denom)

    out = jnp.where(lm, out_left, jnp.where(rm, out_right, out_in))
    lad = jnp.where(lm, lad_left, jnp.where(rm, lad_right, lad_in))
    out_ref[...] = _t(out)
    lad_ref[...] = _t(lad)


def kernel(inputs, unnormalized_widths, unnormalized_heights,
           unnormalized_derivatives, center_x, center_y):
    uw = unnormalized_widths.reshape(_B, _D * _K)
    uh = unnormalized_heights.reshape(_B, _D * _K)
    ud = unnormalized_derivatives.reshape(_B, _D * (_K + 1))

    espec = pl.BlockSpec((_RB, _D), lambda i: (i, 0))
    kspec = lambda k: pl.BlockSpec((_RB, _D * k), lambda i: (i, 0))

    out, lad = pl.pallas_call(
        _body,
        grid=(_G,),
        in_specs=[espec, espec, espec,
                  kspec(_K), kspec(_K), kspec(_K + 1)],
        out_specs=[espec, espec],
        out_shape=[jax.ShapeDtypeStruct((_B, _D), jnp.float32),
                   jax.ShapeDtypeStruct((_B, _D), jnp.float32)],
    )(inputs, center_x, center_y, uw, uh, ud)
    return out, lad
